# baseline (speedup 1.0000x reference)
import numpy as np
import jax
import jax.numpy as jnp
from jax.experimental import pallas as pl
from jax.experimental.pallas import tpu as pltpu
from jax.experimental.pallas import tpu_sc as plsc

D = 1536
B = 4
S = 512
MAX_CHUNKS = 256
MAX_SEQ = 1024
VOCAB = 256
F32 = jnp.float32
BF16 = jnp.bfloat16

_I = False

_INV_SQRT2 = np.float32(1.0 / np.sqrt(2.0))


def _gelu(x):
    return 0.5 * x * (1.0 + jax.lax.erf(x * _INV_SQRT2))


def _onehot(idx, n):
    return (idx[:, None] == jax.lax.broadcasted_iota(jnp.int32, (idx.shape[0], n), 1)).astype(F32)


def _split(x):
    hi = x.astype(BF16)
    lo = (x - hi.astype(F32)).astype(BF16)
    return hi, lo


def _interp_weights(L, out):
    pos = (np.arange(out, dtype=np.float32) + np.float32(0.5)) * np.float32(L / out) - np.float32(0.5)
    pos = np.clip(pos, 0.0, np.float32(L - 1.0))
    lo = np.floor(pos).astype(np.int32)
    hi = np.clip(lo + 1, 0, L - 1)
    w = (pos - lo.astype(np.float32)).astype(np.float32)
    return lo, hi, w


def _build_mbig():
    M = np.zeros((896, 512), np.float32)
    for k in range(511):
        M[k, k + 1] += 1.0
    for off, L in ((512, 255), (768, 127)):
        lo, hi, w = _interp_weights(L, 511)
        for k in range(511):
            M[off + lo[k], k + 1] += np.float32(1.0) - w[k]
            M[off + hi[k], k + 1] += w[k]
    return M


_MBIG = _build_mbig()
_TRIU = np.triu(np.ones((512, 512), np.float32))


def _tables_body(e_ref, dw1_ref, wling_ref, wout_ref, bias_ref, bout_ref,
                 t7h_ref, t7l_ref, ew_ref, gh_ref, gl_ref, nrm_ref, tling_ref):
    j = pl.program_id(0)

    @pl.when(j < 6)
    def _():
        t = jnp.dot(e_ref[...], dw1_ref[0, 0], preferred_element_type=F32) + bias_ref[0]
        hi, lo = _split(t)
        t7h_ref[0] = hi
        t7l_ref[0] = lo

    @pl.when(j == 6)
    def _():
        t = jnp.dot(e_ref[...], wling_ref[...], preferred_element_type=F32) + bias_ref[0]
        tling_ref[...] = t
        hi, lo = _split(t)
        t7h_ref[0] = hi
        t7l_ref[0] = lo

    @pl.when(j == 7)
    def _():
        ew_ref[...] = jnp.dot(e_ref[...], wout_ref[...], preferred_element_type=F32) + bout_ref[...]
        t = tling_ref[...]
        g = jax.lax.dot_general(t, t, (((1,), (1,)), ((), ())), preferred_element_type=F32,
                                precision=jax.lax.Precision.HIGHEST)
        hi, lo = _split(g)
        gh_ref[...] = hi
        gl_ref[...] = lo
        eye = (jax.lax.broadcasted_iota(jnp.int32, (256, 256), 0)
               == jax.lax.broadcasted_iota(jnp.int32, (256, 256), 1))
        nrm_ref[0] = jnp.sqrt(jnp.sum(jnp.where(eye, g, 0.0), axis=0))


def _heads_body(t7h_ref, t7l_ref, ia_ref, ib_ref, dw2_ref, db2_ref, w3_ref, db3_ref, o_ref):
    oa = _onehot(ia_ref[0, 0], 256).astype(BF16)
    ob = _onehot(ib_ref[0, 0], 256).astype(BF16)
    x = (jnp.dot(oa, t7h_ref[0], preferred_element_type=F32)
         + jnp.dot(oa, t7l_ref[0], preferred_element_type=F32)
         + jnp.dot(ob, t7h_ref[1], preferred_element_type=F32)
         + jnp.dot(ob, t7l_ref[1], preferred_element_type=F32))
    h = _gelu(x)
    g = _gelu(jnp.dot(h, dw2_ref[0], preferred_element_type=F32) + db2_ref[0])
    ol = jnp.sum(g * w3_ref[0], axis=1) + db3_ref[pl.program_id(0), 0]
    o = jax.nn.sigmoid(ol)
    o_sh = jnp.concatenate([jnp.zeros((1,), F32), o[:511]])
    o_ref[0, 0, 0] = o_sh


def _segchunks_body(aa_ref, ab_ref, gh_ref, gl_ref, nrm_ref, o_ref, mbig_ref, tri_ref,
                    ids_ref, e_ref, st_ref, cpos_ref, seg_ref, ch_ref):
    oa = _onehot(aa_ref[0, 0], 256)
    ob = _onehot(ab_ref[0, 0], 256)
    oab = oa.astype(BF16)
    ra = (jnp.dot(oab, gh_ref[...], preferred_element_type=F32)
          + jnp.dot(oab, gl_ref[...], preferred_element_type=F32))
    d = jnp.sum(ra * ob, axis=1)
    nv = nrm_ref[0]
    na = jnp.sum(oa * nv[None, :], axis=1)
    nb = jnp.sum(ob * nv[None, :], axis=1)
    cos = d / (na * nb + 1e-8)
    ssum = jnp.dot(cos[None, :], mbig_ref[...], preferred_element_type=F32,
                   precision=jax.lax.Precision.HIGHEST)[0]
    avg = ssum / 3.0
    base = 0.5 * (1.0 - avg)
    lm = (o_ref[0, 0, 0] + o_ref[1, 0, 0] + o_ref[2, 0, 0]) / 3.0
    fb = 0.6 * base + 0.4 * lm
    first = jax.lax.broadcasted_iota(jnp.int32, (512,), 0) == 0
    bits = jnp.where(first, 1.0, (fb > 0.5).astype(F32))
    segf = jnp.dot(bits[None, :], tri_ref[...], preferred_element_type=F32)[0]
    sg = jnp.clip(segf - 1.0, 0.0, 255.0).astype(jnp.int32)
    seg_ref[0, 0] = sg

    ids = ids_ref[0, 0]
    ohs_t = (jax.lax.broadcasted_iota(jnp.int32, (256, 512), 0) == sg[None, :]).astype(F32)
    ohv = _onehot(ids, 256)
    hist = jnp.dot(ohs_t, ohv, preferred_element_type=F32)
    counts = jnp.sum(hist, axis=1)
    sums = jnp.dot(hist, e_ref[...], preferred_element_type=F32)
    mean = sums / jnp.clip(counts, 1.0)[:, None]
    sidx = jnp.clip(counts.astype(jnp.int32), 0, MAX_SEQ - 1)
    ohsz = (sidx[:, None] == jax.lax.broadcasted_iota(jnp.int32, (256, MAX_SEQ), 1)).astype(F32)
    semb = jnp.dot(ohsz, st_ref[...], preferred_element_type=F32)
    ch_ref[0] = mean + semb * (counts > 0.0).astype(F32)[:, None] + cpos_ref[...]


def _cp_body(ch_ref, w1_ref, b1_ref, w2_ref, b2_ref, lng_ref, lnb_ref, wout_ref, cw_ref):
    c = ch_ref[0]
    h = _gelu(jnp.dot(c, w1_ref[...], preferred_element_type=F32) + b1_ref[0])
    h2 = jnp.dot(h, w2_ref[...], preferred_element_type=F32) + b2_ref[0]
    mu = jnp.mean(h2, axis=1, keepdims=True)
    var = jnp.mean((h2 - mu) ** 2, axis=1, keepdims=True)
    co = (h2 - mu) / jnp.sqrt(var + 1e-5) * lng_ref[0] + lnb_ref[0]
    cw_ref[0] = jnp.dot(co, wout_ref[...], preferred_element_type=F32)


def _sc_logits_body(ew_hbm, cw_hbm, ids_hbm, seg_hbm, out_hbm, idx_v, sidx_v, rows_a, rows_b, sem):
    wid = jax.lax.axis_index("s") * 2 + jax.lax.axis_index("c")
    base = wid * 64
    pltpu.sync_copy(ids_hbm.at[pl.ds(base, 64)], idx_v)
    pltpu.sync_copy(seg_hbm.at[pl.ds(base, 64)], sidx_v)
    pltpu.async_copy(ew_hbm.at[idx_v], rows_a, sem).wait()
    pltpu.async_copy(cw_hbm.at[sidx_v], rows_b, sem).wait()

    def body(t, carry):
        for c in range(16):
            sl = pl.ds(c * 16, 16)
            rows_a[t, sl] = rows_a[t, sl] + rows_b[t, sl]
        return carry

    jax.lax.fori_loop(0, 64, body, 0)
    pltpu.sync_copy(rows_a, out_hbm.at[pl.ds(base, 64)])


def kernel(input_ids, embed_table, W_ling, b_ling, dW1, db1, dW2, db2, dW3, db3,
           chunk_pos, size_table, cp_W1, cp_b1, cp_W2, cp_b2, ln_g, ln_b, W_out, b_out):
    ids = input_ids.astype(jnp.int32)
    dW1v = dW1.reshape(3, 2, D, D)
    z = jnp.zeros((D,), F32)
    bias7 = jnp.stack([db1[0], z, db1[1], z, db1[2], z, b_ling]).reshape(7, 1, D)
    bout_r = b_out.reshape(1, VOCAB)
    db2r = db2.reshape(3, 1, D // 2)
    w3r = jnp.transpose(dW3, (0, 2, 1))
    db3r = db3.reshape(3, 1)

    ids_a = ids.reshape(B, 1, S)
    ids_b = jnp.concatenate([ids[:, 1:], ids[:, :1]], axis=1).reshape(B, 1, S)
    e2 = ids[:, ::2]
    e4 = ids[:, ::4]
    pad = ids[:, :1]
    ab_a = jnp.concatenate([ids[:, :511], pad, e2[:, :255], pad, e4[:, :127], pad],
                           axis=1).reshape(B, 1, 896)
    ab_b = jnp.concatenate([ids[:, 1:], pad, e2[:, 1:], pad, e4[:, 1:], pad],
                           axis=1).reshape(B, 1, 896)
    mbig = jnp.asarray(_MBIG)
    tri = jnp.asarray(_TRIU)
    cpos = chunk_pos.reshape(MAX_CHUNKS, D)

    t7h, t7l, ew, g_hi, g_lo, nrm = pl.pallas_call(
        _tables_body,
        grid=(8,),
        in_specs=[
            pl.BlockSpec((VOCAB, D), lambda j: (0, 0)),
            pl.BlockSpec((1, 1, D, D), lambda j: (jnp.minimum(j, 5) // 2, jnp.minimum(j, 5) % 2, 0, 0)),
            pl.BlockSpec((D, D), lambda j: (0, 0)),
            pl.BlockSpec((D, VOCAB), lambda j: (0, 0)),
            pl.BlockSpec((1, 1, D), lambda j: (jnp.minimum(j, 6), 0, 0)),
            pl.BlockSpec((1, VOCAB), lambda j: (0, 0)),
        ],
        out_specs=[
            pl.BlockSpec((1, VOCAB, D), lambda j: (jnp.minimum(j, 6), 0, 0)),
            pl.BlockSpec((1, VOCAB, D), lambda j: (jnp.minimum(j, 6), 0, 0)),
            pl.BlockSpec((VOCAB, VOCAB), lambda j: (0, 0)),
            pl.BlockSpec((VOCAB, VOCAB), lambda j: (0, 0)),
            pl.BlockSpec((VOCAB, VOCAB), lambda j: (0, 0)),
            pl.BlockSpec((1, VOCAB), lambda j: (0, 0)),
        ],
        out_shape=[
            jax.ShapeDtypeStruct((7, VOCAB, D), BF16),
            jax.ShapeDtypeStruct((7, VOCAB, D), BF16),
            jax.ShapeDtypeStruct((VOCAB, VOCAB), F32),
            jax.ShapeDtypeStruct((VOCAB, VOCAB), BF16),
            jax.ShapeDtypeStruct((VOCAB, VOCAB), BF16),
            jax.ShapeDtypeStruct((1, VOCAB), F32),
        ],
        scratch_shapes=[pltpu.VMEM((VOCAB, D), F32)],
        interpret=_I,
    )(embed_table, dW1v, W_ling, W_out, bias7, bout_r)

    o4 = pl.pallas_call(
        _heads_body,
        grid=(3, B),
        in_specs=[
            pl.BlockSpec((2, VOCAB, D), lambda i, b: (i, 0, 0)),
            pl.BlockSpec((2, VOCAB, D), lambda i, b: (i, 0, 0)),
            pl.BlockSpec((1, 1, S), lambda i, b: (b, 0, 0)),
            pl.BlockSpec((1, 1, S), lambda i, b: (b, 0, 0)),
            pl.BlockSpec((1, D, D // 2), lambda i, b: (i, 0, 0)),
            pl.BlockSpec((1, 1, D // 2), lambda i, b: (i, 0, 0)),
            pl.BlockSpec((1, 1, D // 2), lambda i, b: (i, 0, 0)),
            pl.BlockSpec((3, 1), lambda i, b: (0, 0), memory_space=pltpu.SMEM),
        ],
        out_specs=pl.BlockSpec((1, 1, 1, S), lambda i, b: (i, b, 0, 0)),
        out_shape=jax.ShapeDtypeStruct((3, B, 1, S), F32),
        interpret=_I,
    )(t7h, t7l, ids_a, ids_b, dW2, db2r, w3r, db3r)

    seg4, chunks = pl.pallas_call(
        _segchunks_body,
        grid=(B,),
        in_specs=[
            pl.BlockSpec((1, 1, 896), lambda b: (b, 0, 0)),
            pl.BlockSpec((1, 1, 896), lambda b: (b, 0, 0)),
            pl.BlockSpec((VOCAB, VOCAB), lambda b: (0, 0)),
            pl.BlockSpec((VOCAB, VOCAB), lambda b: (0, 0)),
            pl.BlockSpec((1, VOCAB), lambda b: (0, 0)),
            pl.BlockSpec((3, 1, 1, S), lambda b: (0, b, 0, 0)),
            pl.BlockSpec((896, S), lambda b: (0, 0)),
            pl.BlockSpec((S, S), lambda b: (0, 0)),
            pl.BlockSpec((1, 1, S), lambda b: (b, 0, 0)),
            pl.BlockSpec((VOCAB, D), lambda b: (0, 0)),
            pl.BlockSpec((MAX_SEQ, D), lambda b: (0, 0)),
            pl.BlockSpec((MAX_CHUNKS, D), lambda b: (0, 0)),
        ],
        out_specs=[pl.BlockSpec((1, 1, S), lambda b: (b, 0, 0)),
                   pl.BlockSpec((1, MAX_CHUNKS, D), lambda b: (b, 0, 0))],
        out_shape=[jax.ShapeDtypeStruct((B, 1, S), jnp.int32),
                   jax.ShapeDtypeStruct((B, MAX_CHUNKS, D), F32)],
        interpret=_I,
    )(ab_a, ab_b, g_hi, g_lo, nrm, o4, mbig, tri, ids_a, embed_table, size_table, cpos)

    cw = pl.pallas_call(
        _cp_body,
        grid=(B,),
        in_specs=[
            pl.BlockSpec((1, MAX_CHUNKS, D), lambda b: (b, 0, 0)),
            pl.BlockSpec((D, 2 * D), lambda b: (0, 0)),
            pl.BlockSpec((1, 2 * D), lambda b: (0, 0)),
            pl.BlockSpec((2 * D, D), lambda b: (0, 0)),
            pl.BlockSpec((1, D), lambda b: (0, 0)),
            pl.BlockSpec((1, D), lambda b: (0, 0)),
            pl.BlockSpec((1, D), lambda b: (0, 0)),
            pl.BlockSpec((D, VOCAB), lambda b: (0, 0)),
        ],
        out_specs=pl.BlockSpec((1, MAX_CHUNKS, VOCAB), lambda b: (b, 0, 0)),
        out_shape=jax.ShapeDtypeStruct((B, MAX_CHUNKS, VOCAB), F32),
        interpret=_I,
    )(chunks, cp_W1, cp_b1.reshape(1, 2 * D), cp_W2, cp_b2.reshape(1, D),
      ln_g.reshape(1, D), ln_b.reshape(1, D), W_out)

    ids_flat = ids.reshape(B * S)
    seg_flat = (seg4.reshape(B, S)
                + jnp.arange(B, dtype=jnp.int32)[:, None] * MAX_CHUNKS).reshape(B * S)
    cw_flat = cw.reshape(B * MAX_CHUNKS, VOCAB)

    import functools as _ft
    sc_mesh = plsc.VectorSubcoreMesh(core_axis_name="c", subcore_axis_name="s")
    sc_call = _ft.partial(
        pl.kernel,
        mesh=sc_mesh,
        out_type=jax.ShapeDtypeStruct((B * S, VOCAB), F32),
        scratch_types=[
            pltpu.VMEM((64,), jnp.int32),
            pltpu.VMEM((64,), jnp.int32),
            pltpu.VMEM((64, VOCAB), F32),
            pltpu.VMEM((64, VOCAB), F32),
            pltpu.SemaphoreType.DMA,
        ],
    )(_sc_logits_body)
    logits = sc_call(ew, cw_flat, ids_flat, seg_flat)

    return logits.reshape(B, S, VOCAB)


# speedup vs baseline: 1.1360x; 1.1360x over previous
import numpy as np
import jax
import jax.numpy as jnp
from jax.experimental import pallas as pl
from jax.experimental.pallas import tpu as pltpu

D = 1536
B = 4
S = 512
MAX_CHUNKS = 256
MAX_SEQ = 1024
VOCAB = 256
F32 = jnp.float32
BF16 = jnp.bfloat16

_I = False

_INV_SQRT2 = np.float32(1.0 / np.sqrt(2.0))


def _gelu(x):
    return 0.5 * x * (1.0 + jax.lax.erf(x * _INV_SQRT2))


def _onehot(idx, n):
    return (idx[:, None] == jax.lax.broadcasted_iota(jnp.int32, (idx.shape[0], n), 1)).astype(F32)


def _split(x):
    hi = x.astype(BF16)
    lo = (x - hi.astype(F32)).astype(BF16)
    return hi, lo


def _interp_weights(L, out):
    pos = (np.arange(out, dtype=np.float32) + np.float32(0.5)) * np.float32(L / out) - np.float32(0.5)
    pos = np.clip(pos, 0.0, np.float32(L - 1.0))
    lo = np.floor(pos).astype(np.int32)
    hi = np.clip(lo + 1, 0, L - 1)
    w = (pos - lo.astype(np.float32)).astype(np.float32)
    return lo, hi, w


def _build_mbig():
    M = np.zeros((896, 512), np.float32)
    for k in range(511):
        M[k, k + 1] += 1.0
    for off, L in ((512, 255), (768, 127)):
        lo, hi, w = _interp_weights(L, 511)
        for k in range(511):
            M[off + lo[k], k + 1] += np.float32(1.0) - w[k]
            M[off + hi[k], k + 1] += w[k]
    return M


_MBIG = _build_mbig()
_TRIU = np.triu(np.ones((512, 512), np.float32))


def _tables_body(e_ref, dw1_ref, wling_ref, wout_ref, bias_ref, bout_ref,
                 t7h_ref, t7l_ref, ew_ref, gh_ref, gl_ref, nrm_ref, tling_ref):
    j = pl.program_id(0)

    @pl.when(j < 6)
    def _():
        t = jnp.dot(e_ref[...], dw1_ref[0, 0], preferred_element_type=F32) + bias_ref[0]
        hi, lo = _split(t)
        t7h_ref[0] = hi
        t7l_ref[0] = lo

    @pl.when(j == 6)
    def _():
        t = jnp.dot(e_ref[...], wling_ref[...], preferred_element_type=F32) + bias_ref[0]
        tling_ref[...] = t
        hi, lo = _split(t)
        t7h_ref[0] = hi
        t7l_ref[0] = lo

    @pl.when(j == 7)
    def _():
        ew_ref[...] = jnp.dot(e_ref[...], wout_ref[...], preferred_element_type=F32) + bout_ref[...]
        t = tling_ref[...]
        g = jax.lax.dot_general(t, t, (((1,), (1,)), ((), ())), preferred_element_type=F32,
                                precision=jax.lax.Precision.HIGHEST)
        hi, lo = _split(g)
        gh_ref[...] = hi
        gl_ref[...] = lo
        eye = (jax.lax.broadcasted_iota(jnp.int32, (256, 256), 0)
               == jax.lax.broadcasted_iota(jnp.int32, (256, 256), 1))
        nrm_ref[0] = jnp.sqrt(jnp.sum(jnp.where(eye, g, 0.0), axis=0))


def _heads_body(t7h_ref, t7l_ref, ia_ref, ib_ref, dw2_ref, db2_ref, w3_ref, db3_ref, o_ref):
    oa = _onehot(ia_ref[0, 0], 256).astype(BF16)
    ob = _onehot(ib_ref[0, 0], 256).astype(BF16)
    x = (jnp.dot(oa, t7h_ref[0], preferred_element_type=F32)
         + jnp.dot(oa, t7l_ref[0], preferred_element_type=F32)
         + jnp.dot(ob, t7h_ref[1], preferred_element_type=F32)
         + jnp.dot(ob, t7l_ref[1], preferred_element_type=F32))
    h = _gelu(x)
    g = _gelu(jnp.dot(h, dw2_ref[0], preferred_element_type=F32) + db2_ref[0])
    ol = jnp.sum(g * w3_ref[0], axis=1) + db3_ref[pl.program_id(0), 0]
    o = jax.nn.sigmoid(ol)
    o_sh = jnp.concatenate([jnp.zeros((1,), F32), o[:511]])
    o_ref[0, 0, 0] = o_sh


def _segchunks_body(aa_ref, ab_ref, gh_ref, gl_ref, nrm_ref, o_ref, mbig_ref, tri_ref,
                    ids_ref, e_ref, st_ref, cpos_ref, seg_ref, ch_ref):
    oa = _onehot(aa_ref[0, 0], 256)
    ob = _onehot(ab_ref[0, 0], 256)
    oab = oa.astype(BF16)
    ra = (jnp.dot(oab, gh_ref[...], preferred_element_type=F32)
          + jnp.dot(oab, gl_ref[...], preferred_element_type=F32))
    d = jnp.sum(ra * ob, axis=1)
    nv = nrm_ref[0]
    na = jnp.sum(oa * nv[None, :], axis=1)
    nb = jnp.sum(ob * nv[None, :], axis=1)
    cos = d / (na * nb + 1e-8)
    ssum = jnp.dot(cos[None, :], mbig_ref[...], preferred_element_type=F32,
                   precision=jax.lax.Precision.HIGHEST)[0]
    avg = ssum / 3.0
    base = 0.5 * (1.0 - avg)
    lm = (o_ref[0, 0, 0] + o_ref[1, 0, 0] + o_ref[2, 0, 0]) / 3.0
    fb = 0.6 * base + 0.4 * lm
    first = jax.lax.broadcasted_iota(jnp.int32, (512,), 0) == 0
    bits = jnp.where(first, 1.0, (fb > 0.5).astype(F32))
    segf = jnp.dot(bits[None, :], tri_ref[...], preferred_element_type=F32)[0]
    sg = jnp.clip(segf - 1.0, 0.0, 255.0).astype(jnp.int32)
    seg_ref[0, 0] = sg

    ids = ids_ref[0, 0]
    ohs_t = (jax.lax.broadcasted_iota(jnp.int32, (256, 512), 0) == sg[None, :]).astype(F32)
    ohv = _onehot(ids, 256)
    hist = jnp.dot(ohs_t, ohv, preferred_element_type=F32)
    counts = jnp.sum(hist, axis=1)
    sums = jnp.dot(hist, e_ref[...], preferred_element_type=F32)
    mean = sums / jnp.clip(counts, 1.0)[:, None]
    sidx = jnp.clip(counts.astype(jnp.int32), 0, MAX_SEQ - 1)
    ohsz = (sidx[:, None] == jax.lax.broadcasted_iota(jnp.int32, (256, MAX_SEQ), 1)).astype(F32)
    semb = jnp.dot(ohsz, st_ref[...], preferred_element_type=F32)
    ch_ref[0] = mean + semb * (counts > 0.0).astype(F32)[:, None] + cpos_ref[...]


NK = 4
KC = 2 * D // NK


def _cpl_body(ch_ref, w1_ref, b1_ref, w2_ref, b2_ref, lng_ref, lnb_ref, wout_ref,
              ids_ref, seg_ref, ew_ref, out_ref, acc_ref):
    k = pl.program_id(0)
    b = pl.program_id(1)
    c = ch_ref[pl.ds(b, 1)][0]
    hk = _gelu(jnp.dot(c, w1_ref[...], preferred_element_type=F32) + b1_ref[0])
    part = jnp.dot(hk, w2_ref[...], preferred_element_type=F32)

    @pl.when(k == 0)
    def _():
        acc_ref[pl.ds(b, 1)] = part[None]

    @pl.when(k > 0)
    def _():
        acc_ref[pl.ds(b, 1)] = acc_ref[pl.ds(b, 1)] + part[None]

    @pl.when(k == NK - 1)
    def _():
        h2 = acc_ref[pl.ds(b, 1)][0] + b2_ref[0]
        mu = jnp.mean(h2, axis=1, keepdims=True)
        var = jnp.mean((h2 - mu) ** 2, axis=1, keepdims=True)
        co = (h2 - mu) / jnp.sqrt(var + 1e-5) * lng_ref[0] + lnb_ref[0]
        cw = jnp.dot(co, wout_ref[...], preferred_element_type=F32)
        cwh, cwl = _split(cw)
        ohv = _onehot(ids_ref[0, 0], 256)
        ohs = _onehot(seg_ref[0, 0], 256).astype(BF16)
        out_ref[0] = (jnp.dot(ohv, ew_ref[...], preferred_element_type=F32)
                      + jnp.dot(ohs, cwh, preferred_element_type=F32)
                      + jnp.dot(ohs, cwl, preferred_element_type=F32))


def kernel(input_ids, embed_table, W_ling, b_ling, dW1, db1, dW2, db2, dW3, db3,
           chunk_pos, size_table, cp_W1, cp_b1, cp_W2, cp_b2, ln_g, ln_b, W_out, b_out):
    ids = input_ids.astype(jnp.int32)
    dW1v = dW1.reshape(3, 2, D, D)
    z = jnp.zeros((D,), F32)
    bias7 = jnp.stack([db1[0], z, db1[1], z, db1[2], z, b_ling]).reshape(7, 1, D)
    bout_r = b_out.reshape(1, VOCAB)
    db2r = db2.reshape(3, 1, D // 2)
    w3r = jnp.transpose(dW3, (0, 2, 1))
    db3r = db3.reshape(3, 1)

    ids_a = ids.reshape(B, 1, S)
    ids_b = jnp.concatenate([ids[:, 1:], ids[:, :1]], axis=1).reshape(B, 1, S)
    e2 = ids[:, ::2]
    e4 = ids[:, ::4]
    pad = ids[:, :1]
    ab_a = jnp.concatenate([ids[:, :511], pad, e2[:, :255], pad, e4[:, :127], pad],
                           axis=1).reshape(B, 1, 896)
    ab_b = jnp.concatenate([ids[:, 1:], pad, e2[:, 1:], pad, e4[:, 1:], pad],
                           axis=1).reshape(B, 1, 896)
    mbig = jnp.asarray(_MBIG)
    tri = jnp.asarray(_TRIU)
    cpos = chunk_pos.reshape(MAX_CHUNKS, D)

    t7h, t7l, ew, g_hi, g_lo, nrm = pl.pallas_call(
        _tables_body,
        grid=(8,),
        in_specs=[
            pl.BlockSpec((VOCAB, D), lambda j: (0, 0)),
            pl.BlockSpec((1, 1, D, D), lambda j: (jnp.minimum(j, 5) // 2, jnp.minimum(j, 5) % 2, 0, 0)),
            pl.BlockSpec((D, D), lambda j: (0, 0)),
            pl.BlockSpec((D, VOCAB), lambda j: (0, 0)),
            pl.BlockSpec((1, 1, D), lambda j: (jnp.minimum(j, 6), 0, 0)),
            pl.BlockSpec((1, VOCAB), lambda j: (0, 0)),
        ],
        out_specs=[
            pl.BlockSpec((1, VOCAB, D), lambda j: (jnp.minimum(j, 6), 0, 0)),
            pl.BlockSpec((1, VOCAB, D), lambda j: (jnp.minimum(j, 6), 0, 0)),
            pl.BlockSpec((VOCAB, VOCAB), lambda j: (0, 0)),
            pl.BlockSpec((VOCAB, VOCAB), lambda j: (0, 0)),
            pl.BlockSpec((VOCAB, VOCAB), lambda j: (0, 0)),
            pl.BlockSpec((1, VOCAB), lambda j: (0, 0)),
        ],
        out_shape=[
            jax.ShapeDtypeStruct((7, VOCAB, D), BF16),
            jax.ShapeDtypeStruct((7, VOCAB, D), BF16),
            jax.ShapeDtypeStruct((VOCAB, VOCAB), F32),
            jax.ShapeDtypeStruct((VOCAB, VOCAB), BF16),
            jax.ShapeDtypeStruct((VOCAB, VOCAB), BF16),
            jax.ShapeDtypeStruct((1, VOCAB), F32),
        ],
        scratch_shapes=[pltpu.VMEM((VOCAB, D), F32)],
        interpret=_I,
    )(embed_table, dW1v, W_ling, W_out, bias7, bout_r)

    o4 = pl.pallas_call(
        _heads_body,
        grid=(3, B),
        in_specs=[
            pl.BlockSpec((2, VOCAB, D), lambda i, b: (i, 0, 0)),
            pl.BlockSpec((2, VOCAB, D), lambda i, b: (i, 0, 0)),
            pl.BlockSpec((1, 1, S), lambda i, b: (b, 0, 0)),
            pl.BlockSpec((1, 1, S), lambda i, b: (b, 0, 0)),
            pl.BlockSpec((1, D, D // 2), lambda i, b: (i, 0, 0)),
            pl.BlockSpec((1, 1, D // 2), lambda i, b: (i, 0, 0)),
            pl.BlockSpec((1, 1, D // 2), lambda i, b: (i, 0, 0)),
            pl.BlockSpec((3, 1), lambda i, b: (0, 0), memory_space=pltpu.SMEM),
        ],
        out_specs=pl.BlockSpec((1, 1, 1, S), lambda i, b: (i, b, 0, 0)),
        out_shape=jax.ShapeDtypeStruct((3, B, 1, S), F32),
        interpret=_I,
    )(t7h, t7l, ids_a, ids_b, dW2, db2r, w3r, db3r)

    seg4, chunks = pl.pallas_call(
        _segchunks_body,
        grid=(B,),
        in_specs=[
            pl.BlockSpec((1, 1, 896), lambda b: (b, 0, 0)),
            pl.BlockSpec((1, 1, 896), lambda b: (b, 0, 0)),
            pl.BlockSpec((VOCAB, VOCAB), lambda b: (0, 0)),
            pl.BlockSpec((VOCAB, VOCAB), lambda b: (0, 0)),
            pl.BlockSpec((1, VOCAB), lambda b: (0, 0)),
            pl.BlockSpec((3, 1, 1, S), lambda b: (0, b, 0, 0)),
            pl.BlockSpec((896, S), lambda b: (0, 0)),
            pl.BlockSpec((S, S), lambda b: (0, 0)),
            pl.BlockSpec((1, 1, S), lambda b: (b, 0, 0)),
            pl.BlockSpec((VOCAB, D), lambda b: (0, 0)),
            pl.BlockSpec((MAX_SEQ, D), lambda b: (0, 0)),
            pl.BlockSpec((MAX_CHUNKS, D), lambda b: (0, 0)),
        ],
        out_specs=[pl.BlockSpec((1, 1, S), lambda b: (b, 0, 0)),
                   pl.BlockSpec((1, MAX_CHUNKS, D), lambda b: (b, 0, 0))],
        out_shape=[jax.ShapeDtypeStruct((B, 1, S), jnp.int32),
                   jax.ShapeDtypeStruct((B, MAX_CHUNKS, D), F32)],
        interpret=_I,
    )(ab_a, ab_b, g_hi, g_lo, nrm, o4, mbig, tri, ids_a, embed_table, size_table, cpos)

    logits = pl.pallas_call(
        _cpl_body,
        grid=(NK, B),
        in_specs=[
            pl.BlockSpec((B, MAX_CHUNKS, D), lambda k, b: (0, 0, 0)),
            pl.BlockSpec((D, KC), lambda k, b: (0, k)),
            pl.BlockSpec((1, KC), lambda k, b: (0, k)),
            pl.BlockSpec((KC, D), lambda k, b: (k, 0)),
            pl.BlockSpec((1, D), lambda k, b: (0, 0)),
            pl.BlockSpec((1, D), lambda k, b: (0, 0)),
            pl.BlockSpec((1, D), lambda k, b: (0, 0)),
            pl.BlockSpec((D, VOCAB), lambda k, b: (0, 0)),
            pl.BlockSpec((1, 1, S), lambda k, b: (b, 0, 0)),
            pl.BlockSpec((1, 1, S), lambda k, b: (b, 0, 0)),
            pl.BlockSpec((VOCAB, VOCAB), lambda k, b: (0, 0)),
        ],
        out_specs=pl.BlockSpec((1, S, VOCAB), lambda k, b: (b, 0, 0)),
        out_shape=jax.ShapeDtypeStruct((B, S, VOCAB), F32),
        scratch_shapes=[pltpu.VMEM((B, MAX_CHUNKS, D), F32)],
        interpret=_I,
    )(chunks, cp_W1, cp_b1.reshape(1, 2 * D), cp_W2, cp_b2.reshape(1, D),
      ln_g.reshape(1, D), ln_b.reshape(1, D), W_out, ids_a, seg4, ew)

    return logits


# speedup vs baseline: 1.2032x; 1.0591x over previous
import numpy as np
import jax
import jax.numpy as jnp
from jax.experimental import pallas as pl
from jax.experimental.pallas import tpu as pltpu

D = 1536
B = 4
S = 512
MAX_CHUNKS = 256
MAX_SEQ = 1024
VOCAB = 256
F32 = jnp.float32
BF16 = jnp.bfloat16

_I = False

_INV_SQRT2 = np.float32(1.0 / np.sqrt(2.0))


def _gelu(x):
    return 0.5 * x * (1.0 + jax.lax.erf(x * _INV_SQRT2))


def _onehot(idx, n):
    return (idx[:, None] == jax.lax.broadcasted_iota(jnp.int32, (idx.shape[0], n), 1)).astype(F32)


def _split(x):
    hi = x.astype(BF16)
    lo = (x - hi.astype(F32)).astype(BF16)
    return hi, lo


def _interp_weights(L, out):
    pos = (np.arange(out, dtype=np.float32) + np.float32(0.5)) * np.float32(L / out) - np.float32(0.5)
    pos = np.clip(pos, 0.0, np.float32(L - 1.0))
    lo = np.floor(pos).astype(np.int32)
    hi = np.clip(lo + 1, 0, L - 1)
    w = (pos - lo.astype(np.float32)).astype(np.float32)
    return lo, hi, w


def _build_mbig():
    M = np.zeros((896, 512), np.float32)
    for k in range(511):
        M[k, k + 1] += 1.0
    for off, L in ((512, 255), (768, 127)):
        lo, hi, w = _interp_weights(L, 511)
        for k in range(511):
            M[off + lo[k], k + 1] += np.float32(1.0) - w[k]
            M[off + hi[k], k + 1] += w[k]
    return M


_MBIG = _build_mbig()
_TRIU = np.triu(np.ones((512, 512), np.float32))


_KT = 0
_KL = 1
_KH = 2
_KG = 3


def _build_sched():
    seq = []
    seq += [(_KT, 0, 0), (_KT, 0, 1), (_KT, 1, 0), (_KT, 1, 1)]
    heads = [(i, b) for i in range(3) for b in range(4)]
    tabs = [(_KT, n, c) for n in (2, 3, 4, 5) for c in (0, 1)] \
         + [(_KL, 0, 0), (_KL, 0, 1), (_KG, 0, 0)]
    hidx = 0
    order = []
    tpos = 0
    done_tables = 4
    for i, b in heads:
        need_halves = (2 * i + 2) * 2
        while done_tables < need_halves:
            k = tabs[tpos]; tpos += 1; order.append(k)
            if k[0] == _KT:
                done_tables += 1
        order.append((_KH, i, b))
        if tpos < len(tabs):
            k = tabs[tpos]; tpos += 1; order.append(k)
            if k[0] == _KT:
                done_tables += 1
    while tpos < len(tabs):
        order.append(tabs[tpos]); tpos += 1
    return seq + order


_SCHED = _build_sched()
_NSTEP = len(_SCHED)


def _sched_arrays():
    kind = np.array([s[0] for s in _SCHED], np.int32)
    d0 = np.zeros(_NSTEP, np.int32); d1 = np.zeros(_NSTEP, np.int32)
    dc = np.zeros(_NSTEP, np.int32)
    trow = np.zeros(_NSTEP, np.int32); tcol = np.zeros(_NSTEP, np.int32)
    wc = np.zeros(_NSTEP, np.int32)
    bi = np.zeros(_NSTEP, np.int32); bc = np.zeros(_NSTEP, np.int32)
    hi = np.zeros(_NSTEP, np.int32); hb = np.zeros(_NSTEP, np.int32)
    cur = (0, 0, 0); curw = 0; curb = (0, 0); curh = (0, 0)
    for j, (k, a, b) in enumerate(_SCHED):
        if k == _KT:
            cur = (a // 2, a % 2, b); curb = (a, b)
        if k == _KL:
            curw = b; curb = (6, b)
        if k == _KH:
            curh = (a, b)
        d0[j], d1[j], dc[j] = cur
        trow[j], tcol[j] = (a, b) if k in (_KT, _KL) else (0, 0)
        wc[j] = curw
        bi[j], bc[j] = curb
        hi[j], hb[j] = curh
    return np.stack([d0, d1, dc, wc, bi, bc, hi, hb, kind, trow, tcol], axis=1)


def _k12_body(sp_ref, e_ref, dw1_ref, wling_ref, wout_ref, bias_ref, bout_ref,
              ia_ref, ib_ref, dw2_ref, db2_ref, w3_ref, db3_ref,
              o_ref, ew_ref, gh_ref, gl_ref, nrm_ref,
              t7h_s, t7l_s, tling_s):
    j = pl.program_id(0)
    kind = sp_ref[j, 8]
    trow = sp_ref[j, 9]
    tcol = sp_ref[j, 10]
    headi = sp_ref[j, 6]

    @pl.when(kind == _KT)
    def _():
        t = jnp.dot(e_ref[...], dw1_ref[0, 0], preferred_element_type=F32) + bias_ref[0]
        hi_, lo_ = _split(t)
        t7h_s[pl.ds(trow, 1), pl.ds(tcol, 1)] = hi_[None, None]
        t7l_s[pl.ds(trow, 1), pl.ds(tcol, 1)] = lo_[None, None]

    @pl.when(kind == _KL)
    def _():
        t = jnp.dot(e_ref[...], wling_ref[...], preferred_element_type=F32) + bias_ref[0]
        tling_s[pl.ds(tcol, 1)] = t[None]

    @pl.when(kind == _KG)
    def _():
        ew_ref[...] = jnp.dot(e_ref[...], wout_ref[...], preferred_element_type=F32) + bout_ref[...]
        t = jnp.concatenate([tling_s[0], tling_s[1]], axis=1)
        g = jax.lax.dot_general(t, t, (((1,), (1,)), ((), ())), preferred_element_type=F32,
                                precision=jax.lax.Precision.HIGHEST)
        hi_, lo_ = _split(g)
        gh_ref[...] = hi_
        gl_ref[...] = lo_
        eye = (jax.lax.broadcasted_iota(jnp.int32, (256, 256), 0)
               == jax.lax.broadcasted_iota(jnp.int32, (256, 256), 1))
        nrm_ref[0] = jnp.sqrt(jnp.sum(jnp.where(eye, g, 0.0), axis=0))

    @pl.when(kind == _KH)
    def _():
        oa = _onehot(ia_ref[0, 0], 256).astype(BF16)
        ob = _onehot(ib_ref[0, 0], 256).astype(BF16)
        th = t7h_s[pl.ds(2 * headi, 2)]
        tl = t7l_s[pl.ds(2 * headi, 2)]
        xs = []
        for c in range(2):
            xs.append(jnp.dot(oa, th[0, c], preferred_element_type=F32)
                      + jnp.dot(oa, tl[0, c], preferred_element_type=F32)
                      + jnp.dot(ob, th[1, c], preferred_element_type=F32)
                      + jnp.dot(ob, tl[1, c], preferred_element_type=F32))
        x = jnp.concatenate(xs, axis=1)
        h = _gelu(x)
        g = _gelu(jnp.dot(h, dw2_ref[0], preferred_element_type=F32) + db2_ref[0])
        ol = jnp.sum(g * w3_ref[0], axis=1) + db3_ref[headi, 0]
        o = jax.nn.sigmoid(ol)
        o_sh = jnp.concatenate([jnp.zeros((1,), F32), o[:511]])
        o_ref[0, 0, 0] = o_sh


def _segchunks_body(aa_ref, ab_ref, gh_ref, gl_ref, nrm_ref, o_ref, mbig_ref, tri_ref,
                    ids_ref, e_ref, st_ref, cpos_ref, seg_ref, ch_ref):
    oa = _onehot(aa_ref[0, 0], 256)
    ob = _onehot(ab_ref[0, 0], 256)
    oab = oa.astype(BF16)
    ra = (jnp.dot(oab, gh_ref[...], preferred_element_type=F32)
          + jnp.dot(oab, gl_ref[...], preferred_element_type=F32))
    d = jnp.sum(ra * ob, axis=1)
    nv = nrm_ref[0]
    na = jnp.sum(oa * nv[None, :], axis=1)
    nb = jnp.sum(ob * nv[None, :], axis=1)
    cos = d / (na * nb + 1e-8)
    ssum = jnp.dot(cos[None, :], mbig_ref[...], preferred_element_type=F32,
                   precision=jax.lax.Precision.HIGHEST)[0]
    avg = ssum / 3.0
    base = 0.5 * (1.0 - avg)
    lm = (o_ref[0, 0, 0] + o_ref[1, 0, 0] + o_ref[2, 0, 0]) / 3.0
    fb = 0.6 * base + 0.4 * lm
    first = jax.lax.broadcasted_iota(jnp.int32, (512,), 0) == 0
    bits = jnp.where(first, 1.0, (fb > 0.5).astype(F32))
    segf = jnp.dot(bits[None, :], tri_ref[...], preferred_element_type=F32)[0]
    sg = jnp.clip(segf - 1.0, 0.0, 255.0).astype(jnp.int32)
    seg_ref[0, 0] = sg

    ids = ids_ref[0, 0]
    ohs_t = (jax.lax.broadcasted_iota(jnp.int32, (256, 512), 0) == sg[None, :]).astype(F32)
    ohv = _onehot(ids, 256)
    hist = jnp.dot(ohs_t, ohv, preferred_element_type=F32)
    counts = jnp.sum(hist, axis=1)
    sums = jnp.dot(hist, e_ref[...], preferred_element_type=F32)
    mean = sums / jnp.clip(counts, 1.0)[:, None]
    sidx = jnp.clip(counts.astype(jnp.int32), 0, MAX_SEQ - 1)
    ohsz = (sidx[:, None] == jax.lax.broadcasted_iota(jnp.int32, (256, MAX_SEQ), 1)).astype(F32)
    semb = jnp.dot(ohsz, st_ref[...], preferred_element_type=F32)
    ch_ref[0] = mean + semb * (counts > 0.0).astype(F32)[:, None] + cpos_ref[...]


def _cplogits_body(ch_ref, w1_ref, b1_ref, w2_ref, b2_ref, lng_ref, lnb_ref, wout_ref,
                   ids_ref, seg_ref, ew_ref, out_ref):
    c = ch_ref[0]
    h = _gelu(jnp.dot(c, w1_ref[...], preferred_element_type=F32) + b1_ref[0])
    h2 = jnp.dot(h, w2_ref[...], preferred_element_type=F32) + b2_ref[0]
    mu = jnp.mean(h2, axis=1, keepdims=True)
    var = jnp.mean((h2 - mu) ** 2, axis=1, keepdims=True)
    co = (h2 - mu) / jnp.sqrt(var + 1e-5) * lng_ref[0] + lnb_ref[0]
    cw = jnp.dot(co, wout_ref[...], preferred_element_type=F32)
    cwh, cwl = _split(cw)
    ohv = _onehot(ids_ref[0, 0], 256)
    ohs = _onehot(seg_ref[0, 0], 256).astype(BF16)
    out_ref[0] = (jnp.dot(ohv, ew_ref[...], preferred_element_type=F32)
                  + jnp.dot(ohs, cwh, preferred_element_type=F32)
                  + jnp.dot(ohs, cwl, preferred_element_type=F32))


def kernel(input_ids, embed_table, W_ling, b_ling, dW1, db1, dW2, db2, dW3, db3,
           chunk_pos, size_table, cp_W1, cp_b1, cp_W2, cp_b2, ln_g, ln_b, W_out, b_out):
    ids = input_ids.astype(jnp.int32)
    dW1v = dW1.reshape(3, 2, D, D)
    z = jnp.zeros((D,), F32)
    bias7 = jnp.stack([db1[0], z, db1[1], z, db1[2], z, b_ling]).reshape(7, 1, D)
    bout_r = b_out.reshape(1, VOCAB)
    db2r = db2.reshape(3, 1, D // 2)
    w3r = jnp.transpose(dW3, (0, 2, 1))
    db3r = db3.reshape(3, 1)

    ids_a = ids.reshape(B, 1, S)
    ids_b = jnp.concatenate([ids[:, 1:], ids[:, :1]], axis=1).reshape(B, 1, S)
    e2 = ids[:, ::2]
    e4 = ids[:, ::4]
    pad = ids[:, :1]
    ab_a = jnp.concatenate([ids[:, :511], pad, e2[:, :255], pad, e4[:, :127], pad],
                           axis=1).reshape(B, 1, 896)
    ab_b = jnp.concatenate([ids[:, 1:], pad, e2[:, 1:], pad, e4[:, 1:], pad],
                           axis=1).reshape(B, 1, 896)
    mbig = jnp.asarray(_MBIG)
    tri = jnp.asarray(_TRIU)
    cpos = chunk_pos.reshape(MAX_CHUNKS, D)

    spt = jnp.asarray(_sched_arrays())

    grid_spec = pltpu.PrefetchScalarGridSpec(
        num_scalar_prefetch=1,
        grid=(_NSTEP,),
        in_specs=[
            pl.BlockSpec((VOCAB, D), lambda j, sp: (0, 0)),
            pl.BlockSpec((1, 1, D, D // 2), lambda j, sp: (sp[j, 0], sp[j, 1], 0, sp[j, 2])),
            pl.BlockSpec((D, D // 2), lambda j, sp: (0, sp[j, 3])),
            pl.BlockSpec((D, VOCAB), lambda j, sp: (0, 0)),
            pl.BlockSpec((1, 1, D // 2), lambda j, sp: (sp[j, 4], 0, sp[j, 5])),
            pl.BlockSpec((1, VOCAB), lambda j, sp: (0, 0)),
            pl.BlockSpec((1, 1, S), lambda j, sp: (sp[j, 7], 0, 0)),
            pl.BlockSpec((1, 1, S), lambda j, sp: (sp[j, 7], 0, 0)),
            pl.BlockSpec((1, D, D // 2), lambda j, sp: (sp[j, 6], 0, 0)),
            pl.BlockSpec((1, 1, D // 2), lambda j, sp: (sp[j, 6], 0, 0)),
            pl.BlockSpec((1, 1, D // 2), lambda j, sp: (sp[j, 6], 0, 0)),
            pl.BlockSpec((3, 1), lambda j, sp: (0, 0), memory_space=pltpu.SMEM),
        ],
        out_specs=[
            pl.BlockSpec((1, 1, 1, S), lambda j, sp: (sp[j, 6], sp[j, 7], 0, 0)),
            pl.BlockSpec((VOCAB, VOCAB), lambda j, sp: (0, 0)),
            pl.BlockSpec((VOCAB, VOCAB), lambda j, sp: (0, 0)),
            pl.BlockSpec((VOCAB, VOCAB), lambda j, sp: (0, 0)),
            pl.BlockSpec((1, VOCAB), lambda j, sp: (0, 0)),
        ],
        scratch_shapes=[pltpu.VMEM((6, 2, VOCAB, D // 2), BF16),
                        pltpu.VMEM((6, 2, VOCAB, D // 2), BF16),
                        pltpu.VMEM((2, VOCAB, D // 2), F32)],
    )
    o4, ew, g_hi, g_lo, nrm = pl.pallas_call(
        _k12_body,
        grid_spec=grid_spec,
        out_shape=[
            jax.ShapeDtypeStruct((3, B, 1, S), F32),
            jax.ShapeDtypeStruct((VOCAB, VOCAB), F32),
            jax.ShapeDtypeStruct((VOCAB, VOCAB), BF16),
            jax.ShapeDtypeStruct((VOCAB, VOCAB), BF16),
            jax.ShapeDtypeStruct((1, VOCAB), F32),
        ],
        interpret=_I,
    )(spt, embed_table, dW1v, W_ling, W_out, bias7, bout_r,
      ids_a, ids_b, dW2, db2r, w3r, db3r)

    seg4, chunks = pl.pallas_call(
        _segchunks_body,
        grid=(B,),
        in_specs=[
            pl.BlockSpec((1, 1, 896), lambda b: (b, 0, 0)),
            pl.BlockSpec((1, 1, 896), lambda b: (b, 0, 0)),
            pl.BlockSpec((VOCAB, VOCAB), lambda b: (0, 0)),
            pl.BlockSpec((VOCAB, VOCAB), lambda b: (0, 0)),
            pl.BlockSpec((1, VOCAB), lambda b: (0, 0)),
            pl.BlockSpec((3, 1, 1, S), lambda b: (0, b, 0, 0)),
            pl.BlockSpec((896, S), lambda b: (0, 0)),
            pl.BlockSpec((S, S), lambda b: (0, 0)),
            pl.BlockSpec((1, 1, S), lambda b: (b, 0, 0)),
            pl.BlockSpec((VOCAB, D), lambda b: (0, 0)),
            pl.BlockSpec((MAX_SEQ, D), lambda b: (0, 0)),
            pl.BlockSpec((MAX_CHUNKS, D), lambda b: (0, 0)),
        ],
        out_specs=[pl.BlockSpec((1, 1, S), lambda b: (b, 0, 0)),
                   pl.BlockSpec((1, MAX_CHUNKS, D), lambda b: (b, 0, 0))],
        out_shape=[jax.ShapeDtypeStruct((B, 1, S), jnp.int32),
                   jax.ShapeDtypeStruct((B, MAX_CHUNKS, D), F32)],
        interpret=_I,
    )(ab_a, ab_b, g_hi, g_lo, nrm, o4, mbig, tri, ids_a, embed_table, size_table, cpos)

    logits = pl.pallas_call(
        _cplogits_body,
        grid=(B,),
        in_specs=[
            pl.BlockSpec((1, MAX_CHUNKS, D), lambda b: (b, 0, 0)),
            pl.BlockSpec((D, 2 * D), lambda b: (0, 0)),
            pl.BlockSpec((1, 2 * D), lambda b: (0, 0)),
            pl.BlockSpec((2 * D, D), lambda b: (0, 0)),
            pl.BlockSpec((1, D), lambda b: (0, 0)),
            pl.BlockSpec((1, D), lambda b: (0, 0)),
            pl.BlockSpec((1, D), lambda b: (0, 0)),
            pl.BlockSpec((D, VOCAB), lambda b: (0, 0)),
            pl.BlockSpec((1, 1, S), lambda b: (b, 0, 0)),
            pl.BlockSpec((1, 1, S), lambda b: (b, 0, 0)),
            pl.BlockSpec((VOCAB, VOCAB), lambda b: (0, 0)),
        ],
        out_specs=pl.BlockSpec((1, S, VOCAB), lambda b: (b, 0, 0)),
        out_shape=jax.ShapeDtypeStruct((B, S, VOCAB), F32),
        interpret=_I,
    )(chunks, cp_W1, cp_b1.reshape(1, 2 * D), cp_W2, cp_b2.reshape(1, D),
      ln_g.reshape(1, D), ln_b.reshape(1, D), W_out, ids_a, seg4, ew)

    return logits


# speedup vs baseline: 1.2292x; 1.0216x over previous
import numpy as np
import jax
import jax.numpy as jnp
from jax.experimental import pallas as pl
from jax.experimental.pallas import tpu as pltpu

D = 1536
B = 4
S = 512
MAX_CHUNKS = 256
MAX_SEQ = 1024
VOCAB = 256
F32 = jnp.float32
BF16 = jnp.bfloat16


_INV_SQRT2 = np.float32(1.0 / np.sqrt(2.0))


def _gelu(x):
    return 0.5 * x * (1.0 + jax.lax.erf(x * _INV_SQRT2))


def _onehot(idx, n):
    return (idx[:, None] == jax.lax.broadcasted_iota(jnp.int32, (idx.shape[0], n), 1)).astype(F32)


def _split(x):
    hi = x.astype(BF16)
    lo = (x - hi.astype(F32)).astype(BF16)
    return hi, lo


def _interp_weights(L, out):
    pos = (np.arange(out, dtype=np.float32) + np.float32(0.5)) * np.float32(L / out) - np.float32(0.5)
    pos = np.clip(pos, 0.0, np.float32(L - 1.0))
    lo = np.floor(pos).astype(np.int32)
    hi = np.clip(lo + 1, 0, L - 1)
    w = (pos - lo.astype(np.float32)).astype(np.float32)
    return lo, hi, w


def _build_mbig():
    M = np.zeros((896, 512), np.float32)
    for k in range(511):
        M[k, k + 1] += 1.0
    for off, L in ((512, 255), (768, 127)):
        lo, hi, w = _interp_weights(L, 511)
        for k in range(511):
            M[off + lo[k], k + 1] += np.float32(1.0) - w[k]
            M[off + hi[k], k + 1] += w[k]
    return M


_MBIG = _build_mbig()
_TRIU = np.triu(np.ones((512, 512), np.float32))


def _tables_body(e_ref, dw1_ref, wling_ref, wout_ref, bias_ref, bout_ref,
                 t7h_ref, t7l_ref, ew_ref, gh_ref, gl_ref, nrm_ref, tling_ref):
    j = pl.program_id(0)

    @pl.when(j < 6)
    def _():
        t = jnp.dot(e_ref[...], dw1_ref[0, 0], preferred_element_type=F32) + bias_ref[0]
        hi, lo = _split(t)
        t7h_ref[0] = hi
        t7l_ref[0] = lo

    @pl.when(j == 6)
    def _():
        t = jnp.dot(e_ref[...], wling_ref[...], preferred_element_type=F32) + bias_ref[0]
        tling_ref[...] = t
        hi, lo = _split(t)
        t7h_ref[0] = hi
        t7l_ref[0] = lo

    @pl.when(j == 7)
    def _():
        ew_ref[...] = jnp.dot(e_ref[...], wout_ref[...], preferred_element_type=F32) + bout_ref[...]
        t = tling_ref[...]
        g = jax.lax.dot_general(t, t, (((1,), (1,)), ((), ())), preferred_element_type=F32,
                                precision=jax.lax.Precision.HIGHEST)
        hi, lo = _split(g)
        gh_ref[...] = hi
        gl_ref[...] = lo
        eye = (jax.lax.broadcasted_iota(jnp.int32, (256, 256), 0)
               == jax.lax.broadcasted_iota(jnp.int32, (256, 256), 1))
        nrm_ref[0] = jnp.sqrt(jnp.sum(jnp.where(eye, g, 0.0), axis=0))


def _heads_body(t7h_ref, t7l_ref, ia_ref, ib_ref, dw2_ref, db2_ref, w3_ref, db3_ref, o_ref):
    oa = _onehot(ia_ref[0, 0], 256).astype(BF16)
    ob = _onehot(ib_ref[0, 0], 256).astype(BF16)
    x = (jnp.dot(oa, t7h_ref[0], preferred_element_type=F32)
         + jnp.dot(oa, t7l_ref[0], preferred_element_type=F32)
         + jnp.dot(ob, t7h_ref[1], preferred_element_type=F32)
         + jnp.dot(ob, t7l_ref[1], preferred_element_type=F32))
    h = _gelu(x)
    g = _gelu(jnp.dot(h, dw2_ref[0], preferred_element_type=F32) + db2_ref[0])
    ol = jnp.sum(g * w3_ref[0], axis=1) + db3_ref[pl.program_id(0), 0]
    o = jax.nn.sigmoid(ol)
    o_sh = jnp.concatenate([jnp.zeros((1,), F32), o[:511]])
    o_ref[0, 0, 0] = o_sh


def _segchunks_body(aa_ref, ab_ref, gh_ref, gl_ref, nrm_ref, o_ref, mbig_ref, tri_ref,
                    ids_ref, e_ref, st_ref, cpos_ref, seg_ref, ch_ref):
    oa = _onehot(aa_ref[0, 0], 256)
    ob = _onehot(ab_ref[0, 0], 256)
    oab = oa.astype(BF16)
    ra = (jnp.dot(oab, gh_ref[...], preferred_element_type=F32)
          + jnp.dot(oab, gl_ref[...], preferred_element_type=F32))
    d = jnp.sum(ra * ob, axis=1)
    nv = nrm_ref[0]
    na = jnp.sum(oa * nv[None, :], axis=1)
    nb = jnp.sum(ob * nv[None, :], axis=1)
    cos = d / (na * nb + 1e-8)
    ssum = jnp.dot(cos[None, :], mbig_ref[...], preferred_element_type=F32,
                   precision=jax.lax.Precision.HIGHEST)[0]
    avg = ssum / 3.0
    base = 0.5 * (1.0 - avg)
    lm = (o_ref[0, 0, 0] + o_ref[1, 0, 0] + o_ref[2, 0, 0]) / 3.0
    fb = 0.6 * base + 0.4 * lm
    first = jax.lax.broadcasted_iota(jnp.int32, (512,), 0) == 0
    bits = jnp.where(first, 1.0, (fb > 0.5).astype(F32))
    segf = jnp.dot(bits[None, :], tri_ref[...], preferred_element_type=F32)[0]
    sg = jnp.clip(segf - 1.0, 0.0, 255.0).astype(jnp.int32)
    seg_ref[0, 0] = sg

    ids = ids_ref[0, 0]
    ohs_t = (jax.lax.broadcasted_iota(jnp.int32, (256, 512), 0) == sg[None, :]).astype(F32)
    ohv = _onehot(ids, 256)
    hist = jnp.dot(ohs_t, ohv, preferred_element_type=F32)
    counts = jnp.sum(hist, axis=1)
    sums = jnp.dot(hist, e_ref[...], preferred_element_type=F32)
    mean = sums / jnp.clip(counts, 1.0)[:, None]
    sidx = jnp.clip(counts.astype(jnp.int32), 0, MAX_SEQ - 1)
    ohsz = (sidx[:, None] == jax.lax.broadcasted_iota(jnp.int32, (256, MAX_SEQ), 1)).astype(F32)
    semb = jnp.dot(ohsz, st_ref[...], preferred_element_type=F32)
    ch_ref[0] = mean + semb * (counts > 0.0).astype(F32)[:, None] + cpos_ref[...]


def _cplogits_body(ch_ref, w1_ref, b1_ref, w2_ref, b2_ref, lng_ref, lnb_ref, wout_ref,
                   ids_ref, seg_ref, ew_ref, out_ref):
    c = ch_ref[0]
    h = _gelu(jnp.dot(c, w1_ref[...], preferred_element_type=F32) + b1_ref[0])
    h2 = jnp.dot(h, w2_ref[...], preferred_element_type=F32) + b2_ref[0]
    mu = jnp.mean(h2, axis=1, keepdims=True)
    var = jnp.mean((h2 - mu) ** 2, axis=1, keepdims=True)
    co = (h2 - mu) / jnp.sqrt(var + 1e-5) * lng_ref[0] + lnb_ref[0]
    cw = jnp.dot(co, wout_ref[...], preferred_element_type=F32)
    cwh, cwl = _split(cw)
    ohv = _onehot(ids_ref[0, 0], 256)
    ohs = _onehot(seg_ref[0, 0], 256).astype(BF16)
    out_ref[0] = (jnp.dot(ohv, ew_ref[...], preferred_element_type=F32)
                  + jnp.dot(ohs, cwh, preferred_element_type=F32)
                  + jnp.dot(ohs, cwl, preferred_element_type=F32))


def kernel(input_ids, embed_table, W_ling, b_ling, dW1, db1, dW2, db2, dW3, db3,
           chunk_pos, size_table, cp_W1, cp_b1, cp_W2, cp_b2, ln_g, ln_b, W_out, b_out):
    ids = input_ids.astype(jnp.int32)
    dW1v = dW1.reshape(3, 2, D, D)
    z = jnp.zeros((D,), F32)
    bias7 = jnp.stack([db1[0], z, db1[1], z, db1[2], z, b_ling]).reshape(7, 1, D)
    bout_r = b_out.reshape(1, VOCAB)
    db2r = db2.reshape(3, 1, D // 2)
    w3r = jnp.transpose(dW3, (0, 2, 1))
    db3r = db3.reshape(3, 1)

    ids_a = ids.reshape(B, 1, S)
    ids_b = jnp.concatenate([ids[:, 1:], ids[:, :1]], axis=1).reshape(B, 1, S)
    e2 = ids[:, ::2]
    e4 = ids[:, ::4]
    pad = ids[:, :1]
    ab_a = jnp.concatenate([ids[:, :511], pad, e2[:, :255], pad, e4[:, :127], pad],
                           axis=1).reshape(B, 1, 896)
    ab_b = jnp.concatenate([ids[:, 1:], pad, e2[:, 1:], pad, e4[:, 1:], pad],
                           axis=1).reshape(B, 1, 896)
    mbig = jnp.asarray(_MBIG)
    tri = jnp.asarray(_TRIU)
    cpos = chunk_pos.reshape(MAX_CHUNKS, D)

    t7h, t7l, ew, g_hi, g_lo, nrm = pl.pallas_call(
        _tables_body,
        grid=(8,),
        in_specs=[
            pl.BlockSpec((VOCAB, D), lambda j: (0, 0)),
            pl.BlockSpec((1, 1, D, D), lambda j: (jnp.minimum(j, 5) // 2, jnp.minimum(j, 5) % 2, 0, 0)),
            pl.BlockSpec((D, D), lambda j: (0, 0)),
            pl.BlockSpec((D, VOCAB), lambda j: (0, 0)),
            pl.BlockSpec((1, 1, D), lambda j: (jnp.minimum(j, 6), 0, 0)),
            pl.BlockSpec((1, VOCAB), lambda j: (0, 0)),
        ],
        out_specs=[
            pl.BlockSpec((1, VOCAB, D), lambda j: (jnp.minimum(j, 6), 0, 0)),
            pl.BlockSpec((1, VOCAB, D), lambda j: (jnp.minimum(j, 6), 0, 0)),
            pl.BlockSpec((VOCAB, VOCAB), lambda j: (0, 0)),
            pl.BlockSpec((VOCAB, VOCAB), lambda j: (0, 0)),
            pl.BlockSpec((VOCAB, VOCAB), lambda j: (0, 0)),
            pl.BlockSpec((1, VOCAB), lambda j: (0, 0)),
        ],
        out_shape=[
            jax.ShapeDtypeStruct((7, VOCAB, D), BF16),
            jax.ShapeDtypeStruct((7, VOCAB, D), BF16),
            jax.ShapeDtypeStruct((VOCAB, VOCAB), F32),
            jax.ShapeDtypeStruct((VOCAB, VOCAB), BF16),
            jax.ShapeDtypeStruct((VOCAB, VOCAB), BF16),
            jax.ShapeDtypeStruct((1, VOCAB), F32),
        ],
        scratch_shapes=[pltpu.VMEM((VOCAB, D), F32)],
    )(embed_table, dW1v, W_ling, W_out, bias7, bout_r)

    o4 = pl.pallas_call(
        _heads_body,
        grid=(3, B),
        in_specs=[
            pl.BlockSpec((2, VOCAB, D), lambda i, b: (i, 0, 0)),
            pl.BlockSpec((2, VOCAB, D), lambda i, b: (i, 0, 0)),
            pl.BlockSpec((1, 1, S), lambda i, b: (b, 0, 0)),
            pl.BlockSpec((1, 1, S), lambda i, b: (b, 0, 0)),
            pl.BlockSpec((1, D, D // 2), lambda i, b: (i, 0, 0)),
            pl.BlockSpec((1, 1, D // 2), lambda i, b: (i, 0, 0)),
            pl.BlockSpec((1, 1, D // 2), lambda i, b: (i, 0, 0)),
            pl.BlockSpec((3, 1), lambda i, b: (0, 0), memory_space=pltpu.SMEM),
        ],
        out_specs=pl.BlockSpec((1, 1, 1, S), lambda i, b: (i, b, 0, 0)),
        out_shape=jax.ShapeDtypeStruct((3, B, 1, S), F32),
    )(t7h, t7l, ids_a, ids_b, dW2, db2r, w3r, db3r)

    seg4, chunks = pl.pallas_call(
        _segchunks_body,
        grid=(B,),
        in_specs=[
            pl.BlockSpec((1, 1, 896), lambda b: (b, 0, 0)),
            pl.BlockSpec((1, 1, 896), lambda b: (b, 0, 0)),
            pl.BlockSpec((VOCAB, VOCAB), lambda b: (0, 0)),
            pl.BlockSpec((VOCAB, VOCAB), lambda b: (0, 0)),
            pl.BlockSpec((1, VOCAB), lambda b: (0, 0)),
            pl.BlockSpec((3, 1, 1, S), lambda b: (0, b, 0, 0)),
            pl.BlockSpec((896, S), lambda b: (0, 0)),
            pl.BlockSpec((S, S), lambda b: (0, 0)),
            pl.BlockSpec((1, 1, S), lambda b: (b, 0, 0)),
            pl.BlockSpec((VOCAB, D), lambda b: (0, 0)),
            pl.BlockSpec((MAX_SEQ, D), lambda b: (0, 0)),
            pl.BlockSpec((MAX_CHUNKS, D), lambda b: (0, 0)),
        ],
        out_specs=[pl.BlockSpec((1, 1, S), lambda b: (b, 0, 0)),
                   pl.BlockSpec((1, MAX_CHUNKS, D), lambda b: (b, 0, 0))],
        out_shape=[jax.ShapeDtypeStruct((B, 1, S), jnp.int32),
                   jax.ShapeDtypeStruct((B, MAX_CHUNKS, D), F32)],
    )(ab_a, ab_b, g_hi, g_lo, nrm, o4, mbig, tri, ids_a, embed_table, size_table, cpos)

    logits = pl.pallas_call(
        _cplogits_body,
        grid=(B,),
        in_specs=[
            pl.BlockSpec((1, MAX_CHUNKS, D), lambda b: (b, 0, 0)),
            pl.BlockSpec((D, 2 * D), lambda b: (0, 0)),
            pl.BlockSpec((1, 2 * D), lambda b: (0, 0)),
            pl.BlockSpec((2 * D, D), lambda b: (0, 0)),
            pl.BlockSpec((1, D), lambda b: (0, 0)),
            pl.BlockSpec((1, D), lambda b: (0, 0)),
            pl.BlockSpec((1, D), lambda b: (0, 0)),
            pl.BlockSpec((D, VOCAB), lambda b: (0, 0)),
            pl.BlockSpec((1, 1, S), lambda b: (b, 0, 0)),
            pl.BlockSpec((1, 1, S), lambda b: (b, 0, 0)),
            pl.BlockSpec((VOCAB, VOCAB), lambda b: (0, 0)),
        ],
        out_specs=pl.BlockSpec((1, S, VOCAB), lambda b: (b, 0, 0)),
        out_shape=jax.ShapeDtypeStruct((B, S, VOCAB), F32),
    )(chunks, cp_W1, cp_b1.reshape(1, 2 * D), cp_W2, cp_b2.reshape(1, D),
      ln_g.reshape(1, D), ln_b.reshape(1, D), W_out, ids_a, seg4, ew)

    return logits
